# baseline (speedup 1.0000x reference)
import functools

import jax
import jax.numpy as jnp
from jax import lax
from jax.experimental import pallas as pl
from jax.experimental.pallas import tpu as pltpu
from jax.experimental.pallas import tpu_sc as plsc

NC = 2
NS = 16
NW = NC * NS
L = 16

K = 64
KD = 128
RING = 4
ERING = 8


def _worker(base_per_worker):
    cid = lax.axis_index("c")
    sid = lax.axis_index("s")
    wid = cid * NS + sid
    return cid, sid, wid * base_per_worker


def _deg_body(nnodes, nck, dst_hbm, ewp_hbm, zero_hbm, deg_out, dstv, eww,
              rep_a, rep_b, degsh, ssem_a, ssem_b):
    nps = nnodes // NS
    cid, sid, base = _worker(nck * KD)
    pltpu.sync_copy(zero_hbm.at[pl.ds(sid * nps, nps)],
                    degsh.at[pl.ds(sid * nps, nps)])

    pltpu.sync_copy(dst_hbm.at[pl.ds(base, nck * KD)], dstv)
    pltpu.sync_copy(ewp_hbm.at[pl.ds(base, nck * KD + L)], eww)
    plsc.subcore_barrier()

    def build(rep, c):
        @plsc.parallel_loop(0, KD, unroll=2)
        def _(e):
            rep[e, :] = eww[pl.ds(c * KD + e, L)]

    def scat(rep, c, sem):
        pltpu.async_copy(rep, degsh.at[dstv.at[pl.ds(c * KD, KD)]], sem,
                         add=True)

    def scat_wait(rep, c, sem):
        pltpu.make_async_copy(rep, degsh.at[dstv.at[pl.ds(c * KD, KD)]],
                              sem).wait()

    nhalf = nck // 2

    def pair(c2, _):
        ca = 2 * c2
        cb = ca + 1

        @pl.when(c2 > 0)
        def _():
            scat_wait(rep_a, ca - 2, ssem_a)

        build(rep_a, ca)
        scat(rep_a, ca, ssem_a)

        @pl.when(c2 > 0)
        def _():
            scat_wait(rep_b, cb - 2, ssem_b)

        build(rep_b, cb)
        scat(rep_b, cb, ssem_b)
        return 0

    lax.fori_loop(0, nhalf, pair, 0)
    scat_wait(rep_a, nck - 2, ssem_a)
    scat_wait(rep_b, nck - 1, ssem_b)
    plsc.subcore_barrier()
    pltpu.sync_copy(degsh.at[pl.ds(sid * nps, nps)],
                    deg_out.at[cid, pl.ds(sid * nps, nps)])


def _agg_body(nnodes, d, nck, g_hbm, e3_hbm, zero_hbm, out_hbm,
              e3r, r0, r1, r2, r3, f0, f1, gs0, gs1, gs2, gs3, ss0, ss1,
              fsem, accsh):
    nps = nnodes // NS
    cid, sid, cbase = _worker(nck)
    rows = [r0, r1, r2, r3]
    frows = [f0, f1]
    gsems = [gs0, gs1, gs2, gs3]
    ssems = [ss0, ss1]
    pltpu.sync_copy(zero_hbm.at[pl.ds(sid * nps, nps)],
                    accsh.at[pl.ds(sid * nps, nps)])

    ncol = d // L
    k3 = 3 * K

    def scale(cs, fs, es):
        rbf = rows[cs]
        rf = frows[fs]
        esl = e3r.at[es]

        @plsc.parallel_loop(0, K, unroll=2)
        def body(e):
            idx = jnp.zeros((L,), jnp.int32) + (2 * K + e)
            nv = plsc.bitcast(plsc.load_gather(esl, [idx]), jnp.float32)
            for t in range(ncol // 2):
                v = rbf[e, pl.ds(t * 2 * L, 2 * L)]
                a, b = plsc.unpack(v, format=plsc.PackFormat.INTERLEAVED)
                rf[e, pl.ds(t * 2 * L, L)] = a * nv
                rf[e, pl.ds(t * 2 * L + L, L)] = b * nv

    def fetch(c, es):
        pltpu.async_copy(e3_hbm.at[pl.ds((cbase + c) * k3, k3)],
                         e3r.at[es], fsem)

    def fetch_wait(c, es):
        pltpu.make_async_copy(e3_hbm.at[pl.ds((cbase + c) * k3, k3)],
                              e3r.at[es], fsem).wait()

    def gather(cs, es):
        pltpu.async_copy(g_hbm.at[e3r.at[es, pl.ds(0, K)]], rows[cs],
                         gsems[cs])

    def gather_wait(cs, es):
        pltpu.make_async_copy(g_hbm.at[e3r.at[es, pl.ds(0, K)]], rows[cs],
                              gsems[cs]).wait()

    def scat(fs, es):
        pltpu.async_copy(frows[fs], accsh.at[e3r.at[es, pl.ds(K, K)]],
                         ssems[fs], add=True)

    def scat_wait(fs, es):
        pltpu.make_async_copy(frows[fs], accsh.at[e3r.at[es, pl.ds(K, K)]],
                              ssems[fs]).wait()

    def body(i, j, first):
        cs, es = j % RING, j % ERING
        cs2, fs2, es2 = (j - 2) % RING, j % 2, (j - 2) % ERING
        if first:
            if j >= RING:
                scat_wait(j % 2, es)
            fetch(i + RING, (j + RING) % ERING)
        else:
            scat_wait(j % 2, es)

            @pl.when(i + RING < nck)
            def _():
                fetch(i + RING, (j + RING) % ERING)

        fetch_wait(i, es)
        gather(cs, es)
        if not (first and j < 2):
            gather_wait(cs2, es2)
            scale(cs2, fs2, es2)
            scat(fs2, es2)

    for c in range(RING):
        fetch(c, c)
    plsc.subcore_barrier()

    for j in range(ERING):
        body(j, j, True)

    def group(t, _):
        ib = t * ERING
        for j in range(ERING):
            body(ib + j, j, False)
        return 0

    lax.fori_loop(1, nck // ERING, group, 0)

    for j in range(ERING, ERING + 2):
        scat_wait(j % 2, (j - 2) % ERING)
        gather_wait((j - 2) % RING, (j - 2) % ERING)
        scale((j - 2) % RING, j % 2, (j - 2) % ERING)
        scat(j % 2, (j - 2) % ERING)
    for j in range(2):
        scat_wait(j, j)

    plsc.subcore_barrier()
    pltpu.sync_copy(accsh.at[pl.ds(sid * nps, nps)],
                    out_hbm.at[cid, pl.ds(sid * nps, nps)])


def _mm1_body(x_ref, w_ref, degp_ref, g_ref, dinv_ref):
    deg = degp_ref[0, :, 0:1] + degp_ref[1, :, 0:1] + 1.0
    di = lax.rsqrt(deg)
    dinv_ref[:] = di
    h = jnp.dot(x_ref[:], w_ref[:], preferred_element_type=jnp.float32)
    g_ref[:] = di * h


def _combine_body(a_ref, g_ref, dinv_ref, b_ref, o_ref):
    o_ref[:] = jax.nn.relu(
        dinv_ref[:] * (a_ref[0] + a_ref[1] + g_ref[:]) + b_ref[:])


def _combine_mm_body(a_ref, g_ref, dinv_ref, b_ref, w_ref, o_ref):
    o = jax.nn.relu(
        dinv_ref[:] * (a_ref[0] + a_ref[1] + g_ref[:]) + b_ref[:])
    h = jnp.dot(o, w_ref[:], preferred_element_type=jnp.float32)
    o_ref[:] = dinv_ref[:] * h


def _sc_mesh():
    return plsc.VectorSubcoreMesh(core_axis_name="c", subcore_axis_name="s",
                                  num_cores=NC, num_subcores=NS)


_SC_PARAMS = pltpu.CompilerParams(use_tc_tiling_on_sc=False,
                                  needs_layout_passes=False)


def kernel(x, edge_index, edge_weight, W1, b1, W2, b2):
    n, d = x.shape
    e = edge_index.shape[1]
    unit = NW * K * ERING * 2
    epad = ((e + unit - 1) // unit) * unit
    nchunks = epad // K
    nck = nchunks // NW
    nckd = epad // KD // NW
    npad = ((n + NS * 8 - 1) // (NS * 8)) * (NS * 8)
    assert d % L == 0 and nck % ERING == 0 and nckd % 2 == 0

    pad = epad - e
    pad_idx = jnp.arange(pad, dtype=jnp.int32)
    src = jnp.concatenate([edge_index[0].astype(jnp.int32), pad_idx % n])
    dst = jnp.concatenate([edge_index[1].astype(jnp.int32),
                           n + pad_idx % (npad - n)])
    ew = jnp.pad(edge_weight.astype(jnp.float32), (0, pad))
    e3 = jnp.stack([src.reshape(nchunks, K), dst.reshape(nchunks, K),
                    lax.bitcast_convert_type(ew, jnp.int32).reshape(nchunks, K)],
                   axis=1).reshape(-1)
    zeros_nd = jnp.zeros((npad, d), jnp.float32)
    zeros_n16 = jnp.zeros((npad, L), jnp.float32)

    f32 = jnp.float32

    ew_pad = jnp.concatenate([ew, jnp.zeros((L,), jnp.float32)])
    deg_part = pl.kernel(
        functools.partial(_deg_body, npad, nckd),
        out_type=jax.ShapeDtypeStruct((NC, npad, L), f32),
        mesh=_sc_mesh(),
        compiler_params=_SC_PARAMS,
        scratch_types=[
            pltpu.VMEM((nckd * KD,), jnp.int32),
            pltpu.VMEM((nckd * KD + L,), f32),
            pltpu.VMEM((KD, L), f32),
            pltpu.VMEM((KD, L), f32),
            pltpu.VMEM_SHARED((npad, L), f32),
            pltpu.SemaphoreType.DMA,
            pltpu.SemaphoreType.DMA,
        ],
    )(dst, ew_pad, zeros_n16)


    bm = 2000
    grid = n // bm
    row_spec = pl.BlockSpec((bm, d), lambda i: (i, 0))
    agg_spec = pl.BlockSpec((NC, bm, d), lambda i: (0, i, 0))
    col1_spec = pl.BlockSpec((bm, 1), lambda i: (i, 0))
    w_spec = pl.BlockSpec((d, d), lambda i: (0, 0))
    b_spec = pl.BlockSpec((1, d), lambda i: (0, 0))
    out_nd = jax.ShapeDtypeStruct((n, d), f32)

    def sc_agg(gbf):
        out = pl.kernel(
            functools.partial(_agg_body, npad, d, nck),
            out_type=jax.ShapeDtypeStruct((NC, npad, d), f32),
            mesh=_sc_mesh(),
            compiler_params=_SC_PARAMS,
            scratch_types=(
                [pltpu.VMEM((ERING, 3 * K), jnp.int32)]
                + [pltpu.VMEM((K, d), jnp.bfloat16)] * RING
                + [pltpu.VMEM((K, d), f32)] * 2
                + [pltpu.SemaphoreType.DMA] * (RING + 2 + 1)
                + [pltpu.VMEM_SHARED((npad, d), f32)]
            ),
        )(gbf, e3, zeros_nd)
        return out

    degp_spec = pl.BlockSpec((NC, bm, L), lambda i: (0, i, 0))
    g1, dinv = pl.pallas_call(
        _mm1_body, grid=(grid,),
        in_specs=[row_spec, w_spec, degp_spec],
        out_specs=[row_spec, col1_spec],
        out_shape=[out_nd, jax.ShapeDtypeStruct((n, 1), f32)],
    )(x, W1, deg_part)

    def _perm_bf16(g):
        return (g.reshape(n, d // 32, 2, 16).swapaxes(2, 3)
                .reshape(n, d).astype(jnp.bfloat16))

    agg1 = sc_agg(_perm_bf16(g1))

    g2 = pl.pallas_call(
        _combine_mm_body, grid=(grid,),
        in_specs=[agg_spec, row_spec, col1_spec, b_spec, w_spec],
        out_specs=row_spec,
        out_shape=out_nd,
    )(agg1, g1, dinv, b1.reshape(1, d), W2)

    agg2 = sc_agg(_perm_bf16(g2))

    out = pl.pallas_call(
        _combine_body, grid=(grid,),
        in_specs=[agg_spec, row_spec, col1_spec, b_spec],
        out_specs=row_spec,
        out_shape=out_nd,
    )(agg2, g2, dinv, b2.reshape(1, d))
    return out


# speedup vs baseline: 1.0382x; 1.0382x over previous
import functools

import jax
import jax.numpy as jnp
from jax import lax
from jax.experimental import pallas as pl
from jax.experimental.pallas import tpu as pltpu
from jax.experimental.pallas import tpu_sc as plsc

NC = 2
NS = 16
NW = NC * NS
L = 16

K = 64
KD = 128
RING = 5
ERING = 10


def _worker(base_per_worker):
    cid = lax.axis_index("c")
    sid = lax.axis_index("s")
    wid = cid * NS + sid
    return cid, sid, wid * base_per_worker


def _deg_body(nnodes, nck, dst_hbm, ewp_hbm, zero_hbm, deg_out, dstv, eww,
              rep_a, rep_b, degsh, ssem_a, ssem_b):
    nps = nnodes // NS
    cid, sid, base = _worker(nck * KD)
    pltpu.sync_copy(zero_hbm.at[pl.ds(sid * nps, nps)],
                    degsh.at[pl.ds(sid * nps, nps)])

    pltpu.sync_copy(dst_hbm.at[pl.ds(base, nck * KD)], dstv)
    pltpu.sync_copy(ewp_hbm.at[pl.ds(base, nck * KD + L)], eww)
    plsc.subcore_barrier()

    def build(rep, c):
        @plsc.parallel_loop(0, KD, unroll=2)
        def _(e):
            rep[e, :] = eww[pl.ds(c * KD + e, L)]

    def scat(rep, c, sem):
        pltpu.async_copy(rep, degsh.at[dstv.at[pl.ds(c * KD, KD)]], sem,
                         add=True)

    def scat_wait(rep, c, sem):
        pltpu.make_async_copy(rep, degsh.at[dstv.at[pl.ds(c * KD, KD)]],
                              sem).wait()

    nhalf = nck // 2

    def pair(c2, _):
        ca = 2 * c2
        cb = ca + 1

        @pl.when(c2 > 0)
        def _():
            scat_wait(rep_a, ca - 2, ssem_a)

        build(rep_a, ca)
        scat(rep_a, ca, ssem_a)

        @pl.when(c2 > 0)
        def _():
            scat_wait(rep_b, cb - 2, ssem_b)

        build(rep_b, cb)
        scat(rep_b, cb, ssem_b)
        return 0

    lax.fori_loop(0, nhalf, pair, 0)
    scat_wait(rep_a, nck - 2, ssem_a)
    scat_wait(rep_b, nck - 1, ssem_b)
    plsc.subcore_barrier()
    pltpu.sync_copy(degsh.at[pl.ds(sid * nps, nps)],
                    deg_out.at[cid, pl.ds(sid * nps, nps)])


def _agg_body(nnodes, d, nck, g_hbm, e3_hbm, zero_hbm, out_hbm,
              e3r, r0, r1, r2, r3, r4, gs0, gs1, gs2, gs3, gs4,
              ss0, ss1, ss2, ss3, ss4, fsem, accsh):
    nps = nnodes // NS
    cid, sid, cbase = _worker(nck)
    rows = [r0, r1, r2, r3, r4]
    gsems = [gs0, gs1, gs2, gs3, gs4]
    ssems = [ss0, ss1, ss2, ss3, ss4]
    pltpu.sync_copy(zero_hbm.at[pl.ds(sid * nps, nps)],
                    accsh.at[pl.ds(sid * nps, nps)])

    ncol = d // L
    k3 = 3 * K

    def scale(cs, es):
        rws = rows[cs]
        esl = e3r.at[es]

        @plsc.parallel_loop(0, K, unroll=2)
        def body(e):
            idx = jnp.zeros((L,), jnp.int32) + (2 * K + e)
            nv = plsc.bitcast(plsc.load_gather(esl, [idx]), jnp.float32)
            for t in range(ncol):
                sl = pl.ds(t * L, L)
                rws[e, sl] = rws[e, sl] * nv

    def fetch(c, es):
        pltpu.async_copy(e3_hbm.at[pl.ds((cbase + c) * k3, k3)],
                         e3r.at[es], fsem)

    def fetch_wait(c, es):
        pltpu.make_async_copy(e3_hbm.at[pl.ds((cbase + c) * k3, k3)],
                              e3r.at[es], fsem).wait()

    def gather(cs, es):
        pltpu.async_copy(g_hbm.at[e3r.at[es, pl.ds(0, K)]], rows[cs],
                         gsems[cs])

    def gather_wait(cs, es):
        pltpu.make_async_copy(g_hbm.at[e3r.at[es, pl.ds(0, K)]], rows[cs],
                              gsems[cs]).wait()

    def scat(cs, es):
        pltpu.async_copy(rows[cs], accsh.at[e3r.at[es, pl.ds(K, K)]],
                         ssems[cs], add=True)

    def scat_wait(cs, es):
        pltpu.make_async_copy(rows[cs], accsh.at[e3r.at[es, pl.ds(K, K)]],
                              ssems[cs]).wait()

    def body(i, j, first):
        cs, es = j % RING, j % ERING
        cs2, es2 = (j - 2) % RING, (j - 2) % ERING
        if first:
            if j >= RING:
                scat_wait(cs, es)
            fetch(i + RING, (j + RING) % ERING)
        else:
            scat_wait(cs, es)

            @pl.when(i + RING < nck)
            def _():
                fetch(i + RING, (j + RING) % ERING)

        fetch_wait(i, es)
        gather(cs, es)
        if not (first and j < 2):
            gather_wait(cs2, es2)
            scale(cs2, es2)
            scat(cs2, es2)

    for c in range(RING):
        fetch(c, c)
    plsc.subcore_barrier()

    for j in range(ERING):
        body(j, j, True)

    def group(t, _):
        ib = t * ERING
        for j in range(ERING):
            body(ib + j, j, False)
        return 0

    lax.fori_loop(1, nck // ERING, group, 0)

    for j in range(ERING, ERING + 2):
        gather_wait((j - 2) % RING, (j - 2) % ERING)
        scale((j - 2) % RING, (j - 2) % ERING)
        scat((j - 2) % RING, (j - 2) % ERING)
    for j in range(ERING + 2, ERING + 2 + RING):
        scat_wait((j - 2) % RING, (j - 2) % ERING)

    plsc.subcore_barrier()
    pltpu.sync_copy(accsh.at[pl.ds(sid * nps, nps)],
                    out_hbm.at[cid, pl.ds(sid * nps, nps)])


def _mm1_body(x_ref, w_ref, degp_ref, g_ref, dinv_ref):
    deg = degp_ref[0, :, 0:1] + degp_ref[1, :, 0:1] + 1.0
    di = lax.rsqrt(deg)
    dinv_ref[:] = di
    h = jnp.dot(x_ref[:], w_ref[:], preferred_element_type=jnp.float32)
    g_ref[:] = di * h


def _combine_body(a_ref, g_ref, dinv_ref, b_ref, o_ref):
    o_ref[:] = jax.nn.relu(
        dinv_ref[:] * (a_ref[0] + a_ref[1] + g_ref[:]) + b_ref[:])


def _combine_mm_body(a_ref, g_ref, dinv_ref, b_ref, w_ref, o_ref):
    o = jax.nn.relu(
        dinv_ref[:] * (a_ref[0] + a_ref[1] + g_ref[:]) + b_ref[:])
    h = jnp.dot(o, w_ref[:], preferred_element_type=jnp.float32)
    o_ref[:] = dinv_ref[:] * h


def _sc_mesh():
    return plsc.VectorSubcoreMesh(core_axis_name="c", subcore_axis_name="s",
                                  num_cores=NC, num_subcores=NS)


_SC_PARAMS = pltpu.CompilerParams(use_tc_tiling_on_sc=False,
                                  needs_layout_passes=False)


def kernel(x, edge_index, edge_weight, W1, b1, W2, b2):
    n, d = x.shape
    e = edge_index.shape[1]
    unit = NW * K * ERING * 2
    epad = ((e + unit - 1) // unit) * unit
    nchunks = epad // K
    nck = nchunks // NW
    nckd = epad // KD // NW
    npad = ((n + NS * 8 - 1) // (NS * 8)) * (NS * 8)
    assert d % L == 0 and nck % ERING == 0 and nckd % 2 == 0

    pad = epad - e
    pad_idx = jnp.arange(pad, dtype=jnp.int32)
    src = jnp.concatenate([edge_index[0].astype(jnp.int32), pad_idx % n])
    dst = jnp.concatenate([edge_index[1].astype(jnp.int32),
                           n + pad_idx % (npad - n)])
    ew = jnp.pad(edge_weight.astype(jnp.float32), (0, pad))
    e3 = jnp.stack([src.reshape(nchunks, K), dst.reshape(nchunks, K),
                    lax.bitcast_convert_type(ew, jnp.int32).reshape(nchunks, K)],
                   axis=1).reshape(-1)
    zeros_nd = jnp.zeros((npad, d), jnp.float32)
    zeros_n16 = jnp.zeros((npad, L), jnp.float32)

    f32 = jnp.float32

    ew_pad = jnp.concatenate([ew, jnp.zeros((L,), jnp.float32)])
    deg_part = pl.kernel(
        functools.partial(_deg_body, npad, nckd),
        out_type=jax.ShapeDtypeStruct((NC, npad, L), f32),
        mesh=_sc_mesh(),
        compiler_params=_SC_PARAMS,
        scratch_types=[
            pltpu.VMEM((nckd * KD,), jnp.int32),
            pltpu.VMEM((nckd * KD + L,), f32),
            pltpu.VMEM((KD, L), f32),
            pltpu.VMEM((KD, L), f32),
            pltpu.VMEM_SHARED((npad, L), f32),
            pltpu.SemaphoreType.DMA,
            pltpu.SemaphoreType.DMA,
        ],
    )(dst, ew_pad, zeros_n16)


    bm = 2000
    grid = n // bm
    row_spec = pl.BlockSpec((bm, d), lambda i: (i, 0))
    agg_spec = pl.BlockSpec((NC, bm, d), lambda i: (0, i, 0))
    col1_spec = pl.BlockSpec((bm, 1), lambda i: (i, 0))
    w_spec = pl.BlockSpec((d, d), lambda i: (0, 0))
    b_spec = pl.BlockSpec((1, d), lambda i: (0, 0))
    out_nd = jax.ShapeDtypeStruct((n, d), f32)

    def sc_agg(g):
        out = pl.kernel(
            functools.partial(_agg_body, npad, d, nck),
            out_type=jax.ShapeDtypeStruct((NC, npad, d), f32),
            mesh=_sc_mesh(),
            compiler_params=_SC_PARAMS,
            scratch_types=(
                [pltpu.VMEM((ERING, 3 * K), jnp.int32)]
                + [pltpu.VMEM((K, d), f32)] * RING
                + [pltpu.SemaphoreType.DMA] * (2 * RING + 1)
                + [pltpu.VMEM_SHARED((npad, d), f32)]
            ),
        )(g, e3, zeros_nd)
        return out

    degp_spec = pl.BlockSpec((NC, bm, L), lambda i: (0, i, 0))
    g1, dinv = pl.pallas_call(
        _mm1_body, grid=(grid,),
        in_specs=[row_spec, w_spec, degp_spec],
        out_specs=[row_spec, col1_spec],
        out_shape=[out_nd, jax.ShapeDtypeStruct((n, 1), f32)],
    )(x, W1, deg_part)

    agg1 = sc_agg(g1)

    g2 = pl.pallas_call(
        _combine_mm_body, grid=(grid,),
        in_specs=[agg_spec, row_spec, col1_spec, b_spec, w_spec],
        out_specs=row_spec,
        out_shape=out_nd,
    )(agg1, g1, dinv, b1.reshape(1, d), W2)

    agg2 = sc_agg(g2)

    out = pl.pallas_call(
        _combine_body, grid=(grid,),
        in_specs=[agg_spec, row_spec, col1_spec, b_spec],
        out_specs=row_spec,
        out_shape=out_nd,
    )(agg2, g2, dinv, b2.reshape(1, d))
    return out


# speedup vs baseline: 1.0417x; 1.0034x over previous
import functools

import jax
import jax.numpy as jnp
from jax import lax
from jax.experimental import pallas as pl
from jax.experimental.pallas import tpu as pltpu
from jax.experimental.pallas import tpu_sc as plsc

NC = 2
NS = 16
NW = NC * NS
L = 16

K = 64
KD = 128
RING = 4
ERING = 8


def _worker(base_per_worker):
    cid = lax.axis_index("c")
    sid = lax.axis_index("s")
    wid = cid * NS + sid
    return cid, sid, wid * base_per_worker


def _deg_body(nnodes, nck, dst_hbm, ewp_hbm, zero_hbm, deg_out, dstv, eww,
              rep_a, rep_b, degsh, ssem_a, ssem_b):
    nps = nnodes // NS
    cid, sid, base = _worker(nck * KD)
    pltpu.sync_copy(zero_hbm.at[pl.ds(sid * nps, nps)],
                    degsh.at[pl.ds(sid * nps, nps)])

    pltpu.sync_copy(dst_hbm.at[pl.ds(base, nck * KD)], dstv)
    pltpu.sync_copy(ewp_hbm.at[pl.ds(base, nck * KD + L)], eww)
    plsc.subcore_barrier()

    def build(rep, c):
        @plsc.parallel_loop(0, KD, unroll=2)
        def _(e):
            rep[e, :] = eww[pl.ds(c * KD + e, L)]

    def scat(rep, c, sem):
        pltpu.async_copy(rep, degsh.at[dstv.at[pl.ds(c * KD, KD)]], sem,
                         add=True)

    def scat_wait(rep, c, sem):
        pltpu.make_async_copy(rep, degsh.at[dstv.at[pl.ds(c * KD, KD)]],
                              sem).wait()

    nhalf = nck // 2

    def pair(c2, _):
        ca = 2 * c2
        cb = ca + 1

        @pl.when(c2 > 0)
        def _():
            scat_wait(rep_a, ca - 2, ssem_a)

        build(rep_a, ca)
        scat(rep_a, ca, ssem_a)

        @pl.when(c2 > 0)
        def _():
            scat_wait(rep_b, cb - 2, ssem_b)

        build(rep_b, cb)
        scat(rep_b, cb, ssem_b)
        return 0

    lax.fori_loop(0, nhalf, pair, 0)
    scat_wait(rep_a, nck - 2, ssem_a)
    scat_wait(rep_b, nck - 1, ssem_b)
    plsc.subcore_barrier()
    pltpu.sync_copy(degsh.at[pl.ds(sid * nps, nps)],
                    deg_out.at[cid, pl.ds(sid * nps, nps)])


def _agg_body(nnodes, d, nck, g_hbm, e3_hbm, zero_hbm, out_hbm,
              e3r, r0, r1, r2, r3, gs0, gs1, gs2, gs3, ss0, ss1, ss2, ss3,
              fs0, fs1, fs2, fs3, fs4, fs5, fs6, fs7, accsh):
    nps = nnodes // NS
    cid, sid, cbase = _worker(nck)
    rows = [r0, r1, r2, r3]
    gsems = [gs0, gs1, gs2, gs3]
    ssems = [ss0, ss1, ss2, ss3]
    fsems = [fs0, fs1, fs2, fs3, fs4, fs5, fs6, fs7]
    pltpu.sync_copy(zero_hbm.at[pl.ds(sid * nps, nps)],
                    accsh.at[pl.ds(sid * nps, nps)])

    ncol = d // L
    k3 = 3 * K

    def scale(cs, es):
        rws = rows[cs]
        esl = e3r.at[es]

        @plsc.parallel_loop(0, K, unroll=2)
        def body(e):
            idx = jnp.zeros((L,), jnp.int32) + (2 * K + e)
            nv = plsc.bitcast(plsc.load_gather(esl, [idx]), jnp.float32)
            for t in range(ncol):
                sl = pl.ds(t * L, L)
                rws[e, sl] = rws[e, sl] * nv

    def fetch(c, es):
        pltpu.async_copy(e3_hbm.at[pl.ds((cbase + c) * k3, k3)],
                         e3r.at[es], fsems[es])

    def fetch_wait(c, es):
        pltpu.make_async_copy(e3_hbm.at[pl.ds((cbase + c) * k3, k3)],
                              e3r.at[es], fsems[es]).wait()

    def gather(cs, es):
        pltpu.async_copy(g_hbm.at[e3r.at[es, pl.ds(0, K)]], rows[cs],
                         gsems[cs])

    def gather_wait(cs, es):
        pltpu.make_async_copy(g_hbm.at[e3r.at[es, pl.ds(0, K)]], rows[cs],
                              gsems[cs]).wait()

    def scat(cs, es):
        pltpu.async_copy(rows[cs], accsh.at[e3r.at[es, pl.ds(K, K)]],
                         ssems[cs], add=True)

    def scat_wait(cs, es):
        pltpu.make_async_copy(rows[cs], accsh.at[e3r.at[es, pl.ds(K, K)]],
                              ssems[cs]).wait()

    def body(i, j, first):
        cs, es = j % RING, j % ERING
        cs2, es2 = (j - 2) % RING, (j - 2) % ERING
        if first:
            if j >= RING:
                scat_wait(cs, es)
            fetch(i + RING, (j + RING) % ERING)
        else:
            scat_wait(cs, es)

            @pl.when(i + RING < nck)
            def _():
                fetch(i + RING, (j + RING) % ERING)

        fetch_wait(i, es)
        gather(cs, es)
        if not (first and j < 2):
            gather_wait(cs2, es2)
            scale(cs2, es2)
            scat(cs2, es2)

    for c in range(RING):
        fetch(c, c)
    plsc.subcore_barrier()

    for j in range(ERING):
        body(j, j, True)

    def group(t, _):
        ib = t * ERING
        for j in range(ERING):
            body(ib + j, j, False)
        return 0

    lax.fori_loop(1, nck // ERING, group, 0)

    for j in range(ERING, ERING + 2):
        gather_wait((j - 2) % RING, (j - 2) % ERING)
        scale((j - 2) % RING, (j - 2) % ERING)
        scat((j - 2) % RING, (j - 2) % ERING)
    for j in range(ERING + 2, ERING + 2 + RING):
        scat_wait((j - 2) % RING, (j - 2) % ERING)

    plsc.subcore_barrier()
    pltpu.sync_copy(accsh.at[pl.ds(sid * nps, nps)],
                    out_hbm.at[cid, pl.ds(sid * nps, nps)])


def _mm1_body(x_ref, w_ref, degp_ref, g_ref, dinv_ref):
    deg = degp_ref[0, :, 0:1] + degp_ref[1, :, 0:1] + 1.0
    di = lax.rsqrt(deg)
    dinv_ref[:] = di
    h = jnp.dot(x_ref[:], w_ref[:], preferred_element_type=jnp.float32)
    g_ref[:] = di * h


def _combine_body(a_ref, g_ref, dinv_ref, b_ref, o_ref):
    o_ref[:] = jax.nn.relu(
        dinv_ref[:] * (a_ref[0] + a_ref[1] + g_ref[:]) + b_ref[:])


def _combine_mm_body(a_ref, g_ref, dinv_ref, b_ref, w_ref, o_ref):
    o = jax.nn.relu(
        dinv_ref[:] * (a_ref[0] + a_ref[1] + g_ref[:]) + b_ref[:])
    h = jnp.dot(o, w_ref[:], preferred_element_type=jnp.float32)
    o_ref[:] = dinv_ref[:] * h


def _sc_mesh():
    return plsc.VectorSubcoreMesh(core_axis_name="c", subcore_axis_name="s",
                                  num_cores=NC, num_subcores=NS)


_SC_PARAMS = pltpu.CompilerParams(use_tc_tiling_on_sc=False,
                                  needs_layout_passes=False)


def kernel(x, edge_index, edge_weight, W1, b1, W2, b2):
    n, d = x.shape
    e = edge_index.shape[1]
    unit = NW * K * ERING * 2
    epad = ((e + unit - 1) // unit) * unit
    nchunks = epad // K
    nck = nchunks // NW
    nckd = epad // KD // NW
    npad = ((n + NS * 8 - 1) // (NS * 8)) * (NS * 8)
    assert d % L == 0 and nck % ERING == 0 and nckd % 2 == 0

    pad = epad - e
    pad_idx = jnp.arange(pad, dtype=jnp.int32)
    src = jnp.concatenate([edge_index[0].astype(jnp.int32), pad_idx % n])
    dst = jnp.concatenate([edge_index[1].astype(jnp.int32),
                           n + pad_idx % (npad - n)])
    ew = jnp.pad(edge_weight.astype(jnp.float32), (0, pad))
    e3 = jnp.stack([src.reshape(nchunks, K), dst.reshape(nchunks, K),
                    lax.bitcast_convert_type(ew, jnp.int32).reshape(nchunks, K)],
                   axis=1).reshape(-1)
    zeros_nd = jnp.zeros((npad, d), jnp.float32)
    zeros_n16 = jnp.zeros((npad, L), jnp.float32)

    f32 = jnp.float32

    ew_pad = jnp.concatenate([ew, jnp.zeros((L,), jnp.float32)])
    deg_part = pl.kernel(
        functools.partial(_deg_body, npad, nckd),
        out_type=jax.ShapeDtypeStruct((NC, npad, L), f32),
        mesh=_sc_mesh(),
        compiler_params=_SC_PARAMS,
        scratch_types=[
            pltpu.VMEM((nckd * KD,), jnp.int32),
            pltpu.VMEM((nckd * KD + L,), f32),
            pltpu.VMEM((KD, L), f32),
            pltpu.VMEM((KD, L), f32),
            pltpu.VMEM_SHARED((npad, L), f32),
            pltpu.SemaphoreType.DMA,
            pltpu.SemaphoreType.DMA,
        ],
    )(dst, ew_pad, zeros_n16)


    bm = 2000
    grid = n // bm
    row_spec = pl.BlockSpec((bm, d), lambda i: (i, 0))
    agg_spec = pl.BlockSpec((NC, bm, d), lambda i: (0, i, 0))
    col1_spec = pl.BlockSpec((bm, 1), lambda i: (i, 0))
    w_spec = pl.BlockSpec((d, d), lambda i: (0, 0))
    b_spec = pl.BlockSpec((1, d), lambda i: (0, 0))
    out_nd = jax.ShapeDtypeStruct((n, d), f32)

    def sc_agg(g):
        out = pl.kernel(
            functools.partial(_agg_body, npad, d, nck),
            out_type=jax.ShapeDtypeStruct((NC, npad, d), f32),
            mesh=_sc_mesh(),
            compiler_params=_SC_PARAMS,
            scratch_types=(
                [pltpu.VMEM((ERING, 3 * K), jnp.int32)]
                + [pltpu.VMEM((K, d), f32)] * RING
                + [pltpu.SemaphoreType.DMA] * (2 * RING + ERING)
                + [pltpu.VMEM_SHARED((npad, d), f32)]
            ),
        )(g, e3, zeros_nd)
        return out

    degp_spec = pl.BlockSpec((NC, bm, L), lambda i: (0, i, 0))
    g1, dinv = pl.pallas_call(
        _mm1_body, grid=(grid,),
        in_specs=[row_spec, w_spec, degp_spec],
        out_specs=[row_spec, col1_spec],
        out_shape=[out_nd, jax.ShapeDtypeStruct((n, 1), f32)],
    )(x, W1, deg_part)

    agg1 = sc_agg(g1)

    g2 = pl.pallas_call(
        _combine_mm_body, grid=(grid,),
        in_specs=[agg_spec, row_spec, col1_spec, b_spec, w_spec],
        out_specs=row_spec,
        out_shape=out_nd,
    )(agg1, g1, dinv, b1.reshape(1, d), W2)

    agg2 = sc_agg(g2)

    out = pl.pallas_call(
        _combine_body, grid=(grid,),
        in_specs=[agg_spec, row_spec, col1_spec, b_spec],
        out_specs=row_spec,
        out_shape=out_nd,
    )(agg2, g2, dinv, b2.reshape(1, d))
    return out
